# Initial kernel scaffold; baseline (speedup 1.0000x reference)
import functools

import jax
import jax.numpy as jnp
from jax import lax
from jax.experimental import pallas as pl
from jax.experimental.pallas import tpu as pltpu
from jax.experimental.pallas import tpu_sc as plsc

_HARTREE_TO_EV = 27.211386245988
_CUTOFF = 5.0

_info = plsc.get_sparse_core_info()
_NC, _NS, _L = _info.num_cores, _info.num_subcores, _info.num_lanes
_NW = _NC * _NS


def _rsqrt_newton(x):
    xi = plsc.bitcast(x, jnp.int32)
    y = plsc.bitcast(jnp.int32(0x5F3759DF) - (xi >> 1), jnp.float32)
    for _ in range(3):
        y = y * (1.5 - 0.5 * x * y * y)
    return y


def _make_sc_kernel(npa, ppw):
    n_chunks = ppw // _L

    def body(cx_h, cy_h, cz_h, sp_h, a_h, b_h, gsum_out, ehist_out,
             cx_v, cy_v, cz_v, sp_v, a_v, b_v, gsum_v, eh_v):
        wid = lax.axis_index("s") * _NC + lax.axis_index("c")
        base = wid * ppw
        pltpu.sync_copy(cx_h, cx_v)
        pltpu.sync_copy(cy_h, cy_v)
        pltpu.sync_copy(cz_h, cz_v)
        pltpu.sync_copy(sp_h, sp_v)
        pltpu.sync_copy(a_h.at[pl.ds(base, ppw)], a_v)
        pltpu.sync_copy(b_h.at[pl.ds(base, ppw)], b_v)

        zeros = jnp.zeros((_L,), jnp.float32)

        def zero_body(i, carry):
            gsum_v[pl.ds(i * _L, _L)] = zeros
            return carry

        lax.fori_loop(0, npa // _L, zero_body, 0)
        for i in range(256 // _L):
            eh_v[pl.ds(i * _L, _L)] = zeros

        def pair_body(i, carry):
            a16 = a_v[pl.ds(i * _L, _L)]
            b16 = b_v[pl.ds(i * _L, _L)]
            xa = plsc.load_gather(cx_v, [a16])
            ya = plsc.load_gather(cy_v, [a16])
            za = plsc.load_gather(cz_v, [a16])
            xb = plsc.load_gather(cx_v, [b16])
            yb = plsc.load_gather(cy_v, [b16])
            zb = plsc.load_gather(cz_v, [b16])
            spa = plsc.load_gather(sp_v, [a16])
            spb = plsc.load_gather(sp_v, [b16])
            dx = xa - xb
            dy = ya - yb
            dz = za - zb
            d2 = dx * dx + dy * dy + dz * dz + 1e-6
            d = d2 * _rsqrt_newton(d2)
            ge = jnp.where(d <= _CUTOFF, jnp.exp(-d), 0.0)
            glr = jnp.exp(-0.1 * d)
            plsc.addupdate_scatter(eh_v, [spa * 16 + spb], ge)
            plsc.addupdate_scatter(gsum_v, [a16], glr)
            plsc.addupdate_scatter(gsum_v, [b16], glr)
            return carry

        lax.fori_loop(0, n_chunks, pair_body, 0)

        pltpu.sync_copy(gsum_v, gsum_out.at[wid])
        pltpu.sync_copy(eh_v, ehist_out.at[wid])

    return pl.kernel(
        body,
        out_type=[
            jax.ShapeDtypeStruct((_NW, npa), jnp.float32),
            jax.ShapeDtypeStruct((_NW, 256), jnp.float32),
        ],
        mesh=plsc.VectorSubcoreMesh(core_axis_name="c", subcore_axis_name="s"),
        scratch_types=[
            pltpu.VMEM((npa,), jnp.float32),
            pltpu.VMEM((npa,), jnp.float32),
            pltpu.VMEM((npa,), jnp.float32),
            pltpu.VMEM((npa,), jnp.int32),
            pltpu.VMEM((ppw,), jnp.int32),
            pltpu.VMEM((ppw,), jnp.int32),
            pltpu.VMEM((npa,), jnp.float32),
            pltpu.VMEM((256,), jnp.float32),
        ],
    )


def _tc_tail(nrow, n_species, eh_ref, gs_ref, sp_ref, emb_ref, w1_ref, w2_ref,
             e_ref, q_ref):
    ehist = eh_ref[0]
    gsum = gs_ref[0]
    for w in range(1, _NW):
        ehist = ehist + eh_ref[w]
        gsum = gsum + gs_ref[w]
    emb = emb_ref[...]
    a = jnp.dot(emb, w1_ref[...], preferred_element_type=jnp.float32)
    m = lax.dot_general(a, emb, (((1,), (1,)), ((), ())),
                        preferred_element_type=jnp.float32)
    em = jnp.dot(ehist, m, preferred_element_type=jnp.float32)
    row = lax.broadcasted_iota(jnp.int32, (16, 16), 0)
    col = lax.broadcasted_iota(jnp.int32, (16, 16), 1)
    tr = jnp.sum(jnp.where(row == col, em, 0.0))
    e_ref[0, 0] = (jnp.sum(ehist * m) + tr) / _HARTREE_TO_EV
    v = jnp.dot(emb, w2_ref[...], preferred_element_type=jnp.float32)
    sp = sp_ref[...]
    vsel = jnp.zeros((nrow, 128), jnp.float32)
    for z in range(n_species):
        vsel = jnp.where(sp == z, v[z, 0], vsel)
    q_ref[...] = vsel * jnp.tanh(gsum)


def kernel(species, coords, pair_idx, emb, w1, w2):
    n = coords.shape[1]
    p = pair_idx.shape[1]
    z, d = emb.shape
    f32 = jnp.float32

    npa = ((n + 2 + 127) // 128) * 128
    nrow = npa // 128
    ppw = ((p + _NW * _L - 1) // (_NW * _L)) * _L
    pp = ppw * _NW

    c = coords[0]
    cx = jnp.zeros((npa,), f32).at[:n].set(c[:, 0]).at[n + 1].set(1e4)
    cy = jnp.zeros((npa,), f32).at[:n].set(c[:, 1])
    cz = jnp.zeros((npa,), f32).at[:n].set(c[:, 2])
    sp = jnp.zeros((npa,), jnp.int32).at[:n].set(species[0].astype(jnp.int32))
    pad_a = jnp.full((pp - p,), n, jnp.int32)
    pad_b = jnp.full((pp - p,), n + 1, jnp.int32)
    a_idx = jnp.concatenate([pair_idx[0].astype(jnp.int32), pad_a])
    b_idx = jnp.concatenate([pair_idx[1].astype(jnp.int32), pad_b])

    gsums, ehists = _make_sc_kernel(npa, ppw)(cx, cy, cz, sp, a_idx, b_idx)

    emb_p = jnp.zeros((16, 128), f32).at[:z, :d].set(emb)
    w1_p = jnp.zeros((128, 128), f32).at[:d, :d].set(w1)
    w2_p = jnp.zeros((128, 128), f32).at[:d, :1].set(w2)

    energy, charges = pl.pallas_call(
        functools.partial(_tc_tail, nrow, z),
        out_shape=[
            jax.ShapeDtypeStruct((1, 1), f32),
            jax.ShapeDtypeStruct((nrow, 128), f32),
        ],
        out_specs=[
            pl.BlockSpec(memory_space=pltpu.SMEM),
            pl.BlockSpec(memory_space=pltpu.ANY),
        ],
    )(
        ehists.reshape(_NW, 16, 16),
        gsums.reshape(_NW, nrow, 128),
        sp.reshape(nrow, 128),
        emb_p,
        w1_p,
        w2_p,
    )
    return energy.reshape(1), charges.reshape(-1)[:n]


# speedup vs baseline: 69.7333x; 69.7333x over previous
import functools

import jax
import jax.numpy as jnp
from jax import lax
from jax.experimental import pallas as pl
from jax.experimental.pallas import tpu as pltpu
from jax.experimental.pallas import tpu_sc as plsc

_HARTREE_TO_EV = 27.211386245988
_CUTOFF = 5.0

_info = plsc.get_sparse_core_info()
_NC, _NS, _L = _info.num_cores, _info.num_subcores, _info.num_lanes
_NW = _NC * _NS


def _rsqrt_newton(x):
    xi = plsc.bitcast(x, jnp.int32)
    y = plsc.bitcast(jnp.int32(0x5F3759DF) - (xi >> 1), jnp.float32)
    for _ in range(3):
        y = y * (1.5 - 0.5 * x * y * y)
    return y


def _make_sc_kernel(npa, ppw):
    n_chunks = ppw // _L

    def body(cx_h, cy_h, cz_h, sp_h, a_h, b_h, gsum_out, ehist_out,
             cx_v, cy_v, cz_v, sp_v, a_v, b_v, gsum_v, eh_v):
        wid = lax.axis_index("s") * _NC + lax.axis_index("c")
        base = wid * ppw
        pltpu.sync_copy(cx_h, cx_v)
        pltpu.sync_copy(cy_h, cy_v)
        pltpu.sync_copy(cz_h, cz_v)
        pltpu.sync_copy(sp_h, sp_v)
        pltpu.sync_copy(a_h.at[pl.ds(base, ppw)], a_v)
        pltpu.sync_copy(b_h.at[pl.ds(base, ppw)], b_v)

        zeros = jnp.zeros((_L,), jnp.float32)

        def zero_body(i, carry):
            gsum_v[pl.ds(i * _L, _L)] = zeros
            return carry

        lax.fori_loop(0, npa // _L, zero_body, 0)
        for i in range(256 // _L):
            eh_v[pl.ds(i * _L, _L)] = zeros

        def pair_body(i, carry):
            a16 = a_v[pl.ds(i * _L, _L)]
            b16 = b_v[pl.ds(i * _L, _L)]
            xa = plsc.load_gather(cx_v, [a16])
            ya = plsc.load_gather(cy_v, [a16])
            za = plsc.load_gather(cz_v, [a16])
            xb = plsc.load_gather(cx_v, [b16])
            yb = plsc.load_gather(cy_v, [b16])
            zb = plsc.load_gather(cz_v, [b16])
            spa = plsc.load_gather(sp_v, [a16])
            spb = plsc.load_gather(sp_v, [b16])
            dx = xa - xb
            dy = ya - yb
            dz = za - zb
            d2 = dx * dx + dy * dy + dz * dz + 1e-6
            d = d2 * _rsqrt_newton(d2)
            ge = jnp.where(d <= _CUTOFF, jnp.exp(-d), 0.0)
            glr = jnp.exp(-0.1 * d)
            plsc.addupdate_scatter(eh_v, [spa * 16 + spb], ge)
            plsc.addupdate_scatter(gsum_v, [a16], glr)
            plsc.addupdate_scatter(gsum_v, [b16], glr)
            return carry

        lax.fori_loop(0, n_chunks, pair_body, 0)

        pltpu.sync_copy(gsum_v, gsum_out.at[wid])
        pltpu.sync_copy(eh_v, ehist_out.at[wid])

    return pl.kernel(
        body,
        out_type=[
            jax.ShapeDtypeStruct((_NW, npa), jnp.float32),
            jax.ShapeDtypeStruct((_NW, 256), jnp.float32),
        ],
        mesh=plsc.VectorSubcoreMesh(core_axis_name="c", subcore_axis_name="s"),
        compiler_params=pltpu.CompilerParams(needs_layout_passes=False),
        scratch_types=[
            pltpu.VMEM((npa,), jnp.float32),
            pltpu.VMEM((npa,), jnp.float32),
            pltpu.VMEM((npa,), jnp.float32),
            pltpu.VMEM((npa,), jnp.int32),
            pltpu.VMEM((ppw,), jnp.int32),
            pltpu.VMEM((ppw,), jnp.int32),
            pltpu.VMEM((npa,), jnp.float32),
            pltpu.VMEM((256,), jnp.float32),
        ],
    )


def _tc_tail(nrow, n_species, eh_ref, gs_ref, sp_ref, emb_ref, w1_ref, w2_ref,
             e_ref, q_ref):
    ehist = eh_ref[0]
    gsum = gs_ref[0]
    for w in range(1, _NW):
        ehist = ehist + eh_ref[w]
        gsum = gsum + gs_ref[w]
    emb = emb_ref[...]
    a = jnp.dot(emb, w1_ref[...], preferred_element_type=jnp.float32)
    m = lax.dot_general(a, emb, (((1,), (1,)), ((), ())),
                        preferred_element_type=jnp.float32)
    em = jnp.dot(ehist, m, preferred_element_type=jnp.float32)
    row = lax.broadcasted_iota(jnp.int32, (16, 16), 0)
    col = lax.broadcasted_iota(jnp.int32, (16, 16), 1)
    tr = jnp.sum(jnp.where(row == col, em, 0.0))
    e_ref[0, 0] = (jnp.sum(ehist * m) + tr) / _HARTREE_TO_EV
    v = jnp.dot(emb, w2_ref[...], preferred_element_type=jnp.float32)
    sp = sp_ref[...]
    vsel = jnp.zeros((nrow, 128), jnp.float32)
    for z in range(n_species):
        vsel = jnp.where(sp == z, v[z, 0], vsel)
    q_ref[...] = vsel * jnp.tanh(gsum)


def kernel(species, coords, pair_idx, emb, w1, w2):
    n = coords.shape[1]
    p = pair_idx.shape[1]
    z, d = emb.shape
    f32 = jnp.float32

    npa = ((n + 2 + 127) // 128) * 128
    nrow = npa // 128
    ppw = ((p + _NW * _L - 1) // (_NW * _L)) * _L
    pp = ppw * _NW

    c = coords[0]
    cx = jnp.zeros((npa,), f32).at[:n].set(c[:, 0]).at[n + 1].set(1e4)
    cy = jnp.zeros((npa,), f32).at[:n].set(c[:, 1])
    cz = jnp.zeros((npa,), f32).at[:n].set(c[:, 2])
    sp = jnp.zeros((npa,), jnp.int32).at[:n].set(species[0].astype(jnp.int32))
    pad_a = jnp.full((pp - p,), n, jnp.int32)
    pad_b = jnp.full((pp - p,), n + 1, jnp.int32)
    a_idx = jnp.concatenate([pair_idx[0].astype(jnp.int32), pad_a])
    b_idx = jnp.concatenate([pair_idx[1].astype(jnp.int32), pad_b])

    gsums, ehists = _make_sc_kernel(npa, ppw)(cx, cy, cz, sp, a_idx, b_idx)

    emb_p = jnp.zeros((16, 128), f32).at[:z, :d].set(emb)
    w1_p = jnp.zeros((128, 128), f32).at[:d, :d].set(w1)
    w2_p = jnp.zeros((128, 128), f32).at[:d, :1].set(w2)

    energy, charges = pl.pallas_call(
        functools.partial(_tc_tail, nrow, z),
        out_shape=[
            jax.ShapeDtypeStruct((1, 1), f32),
            jax.ShapeDtypeStruct((nrow, 128), f32),
        ],
        out_specs=[
            pl.BlockSpec(memory_space=pltpu.MemorySpace.SMEM),
            pl.BlockSpec(memory_space=pltpu.MemorySpace.VMEM),
        ],
    )(
        ehists.reshape(_NW, 16, 16),
        gsums.reshape(_NW, nrow, 128),
        sp.reshape(nrow, 128),
        emb_p,
        w1_p,
        w2_p,
    )
    return energy.reshape(1), charges.reshape(-1)[:n]
